# Initial kernel scaffold; baseline (speedup 1.0000x reference)
import jax
import jax.numpy as jnp
from jax.experimental import pallas as pl


def kernel(v, batch_idx, mask, count, rank_scores, W1, b1, W2, b2):
    raise NotImplementedError("write your pallas kernel here")


# speedup vs baseline: 17.6375x; 17.6375x over previous
import functools

import jax
import jax.numpy as jnp
import numpy as np
from jax import lax
from jax.experimental import pallas as pl
from jax.experimental.pallas import tpu as pltpu
from jax.experimental.pallas import tpu_sc as plsc

D_VEC = 128
L_POS = 50
K_SET = 16
N_CH = 4096
HID = 256

_NEG = float(np.finfo(np.float32).min)
_SEL_C = 256
_MLP_C = 256
_NCORE = 2
_NW = 32


def _select_body(s_ref, m_ref, i_ref, g_ref, n_ref):
    ms = jnp.where(m_ref[...] > 0, s_ref[...], _NEG)
    c = ms.shape[1]
    jio = lax.broadcasted_iota(jnp.int32, (L_POS, c), 0)
    rank = jnp.zeros((L_POS, c), jnp.float32)
    for i in range(L_POS):
        ri = ms[i:i + 1, :]
        gt = ri > ms
        tie = (ri == ms) & (i < jio)
        rank = rank + jnp.where(gt | tie, 1.0, 0.0)
    sel = (m_ref[...] > 0) & (rank < float(K_SET))
    self_ = jnp.where(sel, 1.0, 0.0)
    a = lax.broadcasted_iota(jnp.int32, (L_POS, L_POS), 0)
    b = lax.broadcasted_iota(jnp.int32, (L_POS, L_POS), 1)
    tri = jnp.where(a > b, 1.0, 0.0)
    slot = jnp.dot(tri, self_, preferred_element_type=jnp.float32)
    n_ref[...] = jnp.sum(self_, axis=0, keepdims=True)
    idxf = i_ref[...]
    for s in range(K_SET):
        on = self_ * jnp.where(slot == float(s), 1.0, 0.0)
        g_ref[s:s + 1, :] = jnp.sum(idxf * on, axis=0,
                                    keepdims=True).astype(jnp.int32)


def _select(sT, mT, iT):
    grid = N_CH // _SEL_C
    return pl.pallas_call(
        _select_body,
        grid=(grid,),
        in_specs=[pl.BlockSpec((L_POS, _SEL_C), lambda b: (0, b))
                  for _ in range(3)],
        out_specs=[pl.BlockSpec((K_SET, _SEL_C), lambda b: (0, b)),
                   pl.BlockSpec((1, _SEL_C), lambda b: (0, b))],
        out_shape=[jax.ShapeDtypeStruct((K_SET, N_CH), jnp.int32),
                   jax.ShapeDtypeStruct((1, N_CH), jnp.float32)],
    )(sT, mT, iT)


def _gather(idx2d, v):
    mesh = plsc.VectorSubcoreMesh(core_axis_name="c", subcore_axis_name="s")
    rows_per_w = (N_CH * K_SET) // _NW // 128

    @functools.partial(
        pl.kernel, mesh=mesh,
        out_type=jax.ShapeDtypeStruct((N_CH * K_SET, D_VEC), jnp.float32),
        scratch_types=[
            pltpu.VMEM((rows_per_w, 128), jnp.int32),
            pltpu.VMEM((128, D_VEC), jnp.float32),
            pltpu.VMEM((128, D_VEC), jnp.float32),
            pltpu.SemaphoreType.DMA,
            pltpu.SemaphoreType.DMA,
        ])
    def k(idx_hbm, table_hbm, out_hbm, idx_v, b0, b1, s0, s1):
        wid = lax.axis_index("s") * _NCORE + lax.axis_index("c")
        p0 = wid * rows_per_w
        pltpu.sync_copy(idx_hbm.at[pl.ds(p0, rows_per_w)], idx_v)
        bufs = (b0, b1)
        sems = (s0, s1)
        cp = pltpu.async_copy(table_hbm.at[idx_v.at[0]], b0, s0)
        for j in range(1, rows_per_w + 1):
            nxt = None
            if j < rows_per_w:
                nxt = pltpu.async_copy(table_hbm.at[idx_v.at[j]],
                                       bufs[j % 2], sems[j % 2])
            cp.wait()
            pltpu.sync_copy(bufs[(j - 1) % 2],
                            out_hbm.at[pl.ds((p0 + (j - 1)) * 128, 128)])
            cp = nxt

    return k(idx2d, v)


def _mlp_body(p_ref, n_ref, c_ref, w1_ref, wc_ref, b1_ref, w2_ref, b2_ref,
              o_ref):
    h = jnp.log1p(c_ref[...]) * wc_ref[...] + b1_ref[...]
    ns = n_ref[...]
    for s in range(K_SET):
        m = jnp.where(ns > float(s), 1.0, 0.0)
        xs = p_ref[s] * m
        h = h + jnp.dot(xs, w1_ref[s], preferred_element_type=jnp.float32)
    act = 0.5 * h * (1.0 + lax.erf(h * np.float32(1.0 / np.sqrt(2.0))))
    o_ref[...] = (jnp.dot(act, w2_ref[...],
                          preferred_element_type=jnp.float32) + b2_ref[...])


def _mlp(packedT, nsel_c, cnt_c, W1m, w1c, b1r, W2, b2r):
    grid = N_CH // _MLP_C
    return pl.pallas_call(
        _mlp_body,
        grid=(grid,),
        in_specs=[
            pl.BlockSpec((K_SET, _MLP_C, D_VEC), lambda b: (0, b, 0)),
            pl.BlockSpec((_MLP_C, 1), lambda b: (b, 0)),
            pl.BlockSpec((_MLP_C, 1), lambda b: (b, 0)),
            pl.BlockSpec((K_SET, D_VEC, HID), lambda b: (0, 0, 0)),
            pl.BlockSpec((1, HID), lambda b: (0, 0)),
            pl.BlockSpec((1, HID), lambda b: (0, 0)),
            pl.BlockSpec((HID, D_VEC), lambda b: (0, 0)),
            pl.BlockSpec((1, D_VEC), lambda b: (0, 0)),
        ],
        out_specs=pl.BlockSpec((_MLP_C, D_VEC), lambda b: (b, 0)),
        out_shape=jax.ShapeDtypeStruct((N_CH, D_VEC), jnp.float32),
    )(packedT, nsel_c, cnt_c, W1m, w1c, b1r, W2, b2r)


def kernel(v, batch_idx, mask, count, rank_scores, W1, b1, W2, b2):
    sT = rank_scores.T
    mT = mask.T.astype(jnp.float32)
    iT = batch_idx.astype(jnp.float32).T
    gT, nselT = _select(sT, mT, iT)
    idx2d = gT.reshape((K_SET * N_CH) // 128, 128)
    packed = _gather(idx2d, v)
    packedT = packed.reshape(K_SET, N_CH, D_VEC)
    nsel_c = nselT.reshape(N_CH, 1)
    cnt_c = count.reshape(N_CH, 1)
    W1m = W1[:K_SET * D_VEC].reshape(K_SET, D_VEC, HID)
    w1c = W1[K_SET * D_VEC:].reshape(1, HID)
    return _mlp(packedT, nsel_c, cnt_c, W1m, w1c, b1.reshape(1, HID),
                W2, b2.reshape(1, D_VEC))


# speedup vs baseline: 17.7035x; 1.0037x over previous
import functools

import jax
import jax.numpy as jnp
import numpy as np
from jax import lax
from jax.experimental import pallas as pl
from jax.experimental.pallas import tpu as pltpu
from jax.experimental.pallas import tpu_sc as plsc

D_VEC = 128
L_POS = 50
K_SET = 16
N_CH = 4096
HID = 256

_NEG = float(np.finfo(np.float32).min)
_SEL_C = 256
_MLP_C = 256
_NCORE = 2
_NW = 32


def _select_body(s_ref, m_ref, i_ref, g_ref, n_ref):
    ms = jnp.where(m_ref[...] > 0, s_ref[...], _NEG)
    c = ms.shape[1]
    jio = lax.broadcasted_iota(jnp.int32, (L_POS, c), 0)
    rank = jnp.zeros((L_POS, c), jnp.float32)
    for i in range(L_POS):
        ri = ms[i:i + 1, :]
        gt = ri > ms
        tie = (ri == ms) & (i < jio)
        rank = rank + jnp.where(gt | tie, 1.0, 0.0)
    sel = (m_ref[...] > 0) & (rank < float(K_SET))
    self_ = jnp.where(sel, 1.0, 0.0)
    a = lax.broadcasted_iota(jnp.int32, (L_POS, L_POS), 0)
    b = lax.broadcasted_iota(jnp.int32, (L_POS, L_POS), 1)
    tri = jnp.where(a > b, 1.0, 0.0)
    slot = jnp.dot(tri, self_, preferred_element_type=jnp.float32)
    n_ref[...] = jnp.sum(self_, axis=0, keepdims=True)
    idxf = i_ref[...]
    for s in range(K_SET):
        on = self_ * jnp.where(slot == float(s), 1.0, 0.0)
        g_ref[s:s + 1, :] = jnp.sum(idxf * on, axis=0,
                                    keepdims=True).astype(jnp.int32)


def _select(sT, mT, iT):
    grid = N_CH // _SEL_C
    return pl.pallas_call(
        _select_body,
        grid=(grid,),
        in_specs=[pl.BlockSpec((L_POS, _SEL_C), lambda b: (0, b))
                  for _ in range(3)],
        out_specs=[pl.BlockSpec((K_SET, _SEL_C), lambda b: (0, b)),
                   pl.BlockSpec((1, _SEL_C), lambda b: (0, b))],
        out_shape=[jax.ShapeDtypeStruct((K_SET, N_CH), jnp.int32),
                   jax.ShapeDtypeStruct((1, N_CH), jnp.float32)],
    )(sT, mT, iT)


def _gather(idx2d, v):
    mesh = plsc.VectorSubcoreMesh(core_axis_name="c", subcore_axis_name="s")
    rows_per_w = (N_CH * K_SET) // _NW // 128

    @functools.partial(
        pl.kernel, mesh=mesh,
        out_type=jax.ShapeDtypeStruct((N_CH * K_SET, D_VEC), jnp.float32),
        scratch_types=[
            pltpu.VMEM((rows_per_w, 128), jnp.int32),
            pltpu.VMEM((128, D_VEC), jnp.float32),
            pltpu.VMEM((128, D_VEC), jnp.float32),
            pltpu.VMEM((128, D_VEC), jnp.float32),
            pltpu.VMEM((128, D_VEC), jnp.float32),
            pltpu.SemaphoreType.DMA,
            pltpu.SemaphoreType.DMA,
            pltpu.SemaphoreType.DMA,
            pltpu.SemaphoreType.DMA,
            pltpu.SemaphoreType.DMA,
            pltpu.SemaphoreType.DMA,
            pltpu.SemaphoreType.DMA,
            pltpu.SemaphoreType.DMA,
        ])
    def k(idx_hbm, table_hbm, out_hbm, idx_v,
          b0, b1, b2, b3, g0, g1, g2, g3, w0, w1, w2, w3):
        wid = lax.axis_index("s") * _NCORE + lax.axis_index("c")
        p0 = wid * rows_per_w
        pltpu.sync_copy(idx_hbm.at[pl.ds(p0, rows_per_w)], idx_v)
        bufs = (b0, b1, b2, b3)
        gsems = (g0, g1, g2, g3)
        wsems = (w0, w1, w2, w3)
        nb = 4
        gcp = [None] * nb
        wcp = [None] * nb
        for j in range(rows_per_w + 1):
            if j < rows_per_w:
                b = j % nb
                if wcp[b] is not None:
                    wcp[b].wait()
                gcp[b] = pltpu.async_copy(table_hbm.at[idx_v.at[j]],
                                          bufs[b], gsems[b])
            if j >= 1:
                b = (j - 1) % nb
                gcp[b].wait()
                wcp[b] = pltpu.async_copy(
                    bufs[b], out_hbm.at[pl.ds((p0 + (j - 1)) * 128, 128)],
                    wsems[b])
        for b in range(nb):
            if wcp[b] is not None:
                wcp[b].wait()

    return k(idx2d, v)


def _mlp_body(p_ref, n_ref, c_ref, w1_ref, wc_ref, b1_ref, w2_ref, b2_ref,
              o_ref):
    h = jnp.log1p(c_ref[...]) * wc_ref[...] + b1_ref[...]
    ns = n_ref[...]
    for s in range(K_SET):
        m = jnp.where(ns > float(s), 1.0, 0.0)
        xs = p_ref[s] * m
        h = h + jnp.dot(xs, w1_ref[s], preferred_element_type=jnp.float32)
    act = 0.5 * h * (1.0 + lax.erf(h * np.float32(1.0 / np.sqrt(2.0))))
    o_ref[...] = (jnp.dot(act, w2_ref[...],
                          preferred_element_type=jnp.float32) + b2_ref[...])


def _mlp(packedT, nsel_c, cnt_c, W1m, w1c, b1r, W2, b2r):
    grid = N_CH // _MLP_C
    return pl.pallas_call(
        _mlp_body,
        grid=(grid,),
        in_specs=[
            pl.BlockSpec((K_SET, _MLP_C, D_VEC), lambda b: (0, b, 0)),
            pl.BlockSpec((_MLP_C, 1), lambda b: (b, 0)),
            pl.BlockSpec((_MLP_C, 1), lambda b: (b, 0)),
            pl.BlockSpec((K_SET, D_VEC, HID), lambda b: (0, 0, 0)),
            pl.BlockSpec((1, HID), lambda b: (0, 0)),
            pl.BlockSpec((1, HID), lambda b: (0, 0)),
            pl.BlockSpec((HID, D_VEC), lambda b: (0, 0)),
            pl.BlockSpec((1, D_VEC), lambda b: (0, 0)),
        ],
        out_specs=pl.BlockSpec((_MLP_C, D_VEC), lambda b: (b, 0)),
        out_shape=jax.ShapeDtypeStruct((N_CH, D_VEC), jnp.float32),
    )(packedT, nsel_c, cnt_c, W1m, w1c, b1r, W2, b2r)


def kernel(v, batch_idx, mask, count, rank_scores, W1, b1, W2, b2):
    sT = rank_scores.T
    mT = mask.T.astype(jnp.float32)
    iT = batch_idx.astype(jnp.float32).T
    gT, nselT = _select(sT, mT, iT)
    idx2d = gT.reshape((K_SET * N_CH) // 128, 128)
    packed = _gather(idx2d, v)
    packedT = packed.reshape(K_SET, N_CH, D_VEC)
    nsel_c = nselT.reshape(N_CH, 1)
    cnt_c = count.reshape(N_CH, 1)
    W1m = W1[:K_SET * D_VEC].reshape(K_SET, D_VEC, HID)
    w1c = W1[K_SET * D_VEC:].reshape(1, HID)
    return _mlp(packedT, nsel_c, cnt_c, W1m, w1c, b1.reshape(1, HID),
                W2, b2.reshape(1, D_VEC))


# speedup vs baseline: 18.1070x; 1.0228x over previous
import functools

import jax
import jax.numpy as jnp
import numpy as np
from jax import lax
from jax.experimental import pallas as pl
from jax.experimental.pallas import tpu as pltpu
from jax.experimental.pallas import tpu_sc as plsc

D_VEC = 128
L_POS = 50
K_SET = 16
N_CH = 4096
HID = 256

_NEG = float(np.finfo(np.float32).min)
_SEL_C = 256
_MLP_C = 256
_NCORE = 2
_NW = 32


def _select_body(s_ref, m_ref, i_ref, g_ref, n_ref):
    ms = jnp.where(m_ref[...] > 0, s_ref[...], _NEG)
    c = ms.shape[1]
    jio = lax.broadcasted_iota(jnp.int32, (L_POS, c), 0)
    rank = jnp.zeros((L_POS, c), jnp.float32)
    for i in range(L_POS):
        ri = ms[i:i + 1, :]
        gt = ri > ms
        tie = (ri == ms) & (i < jio)
        rank = rank + jnp.where(gt | tie, 1.0, 0.0)
    sel = (m_ref[...] > 0) & (rank < float(K_SET))
    self_ = jnp.where(sel, 1.0, 0.0)
    a = lax.broadcasted_iota(jnp.int32, (L_POS, L_POS), 0)
    b = lax.broadcasted_iota(jnp.int32, (L_POS, L_POS), 1)
    tri = jnp.where(a > b, 1.0, 0.0)
    slot = jnp.dot(tri, self_, preferred_element_type=jnp.float32)
    n_ref[...] = jnp.sum(self_, axis=0, keepdims=True)
    idxf = i_ref[...]
    for s in range(K_SET):
        on = self_ * jnp.where(slot == float(s), 1.0, 0.0)
        g_ref[s:s + 1, :] = jnp.sum(idxf * on, axis=0,
                                    keepdims=True).astype(jnp.int32)


def _select(sT, mT, iT):
    grid = N_CH // _SEL_C
    return pl.pallas_call(
        _select_body,
        grid=(grid,),
        in_specs=[pl.BlockSpec((L_POS, _SEL_C), lambda b: (0, b))
                  for _ in range(3)],
        out_specs=[pl.BlockSpec((K_SET, _SEL_C), lambda b: (0, b)),
                   pl.BlockSpec((1, _SEL_C), lambda b: (0, b))],
        out_shape=[jax.ShapeDtypeStruct((K_SET, N_CH), jnp.int32),
                   jax.ShapeDtypeStruct((1, N_CH), jnp.float32)],
    )(sT, mT, iT)


def _gather(idx2d, v):
    mesh = plsc.VectorSubcoreMesh(core_axis_name="c", subcore_axis_name="s")
    rows_per_w = (N_CH * K_SET) // _NW // 128

    @functools.partial(
        pl.kernel, mesh=mesh,
        out_type=jax.ShapeDtypeStruct((N_CH, K_SET * D_VEC), jnp.float32),
        scratch_types=[
            pltpu.VMEM((rows_per_w, 128), jnp.int32),
            pltpu.VMEM((128, D_VEC), jnp.float32),
            pltpu.VMEM((128, D_VEC), jnp.float32),
            pltpu.VMEM((128, D_VEC), jnp.float32),
            pltpu.VMEM((128, D_VEC), jnp.float32),
            pltpu.SemaphoreType.DMA,
            pltpu.SemaphoreType.DMA,
            pltpu.SemaphoreType.DMA,
            pltpu.SemaphoreType.DMA,
            pltpu.SemaphoreType.DMA,
            pltpu.SemaphoreType.DMA,
            pltpu.SemaphoreType.DMA,
            pltpu.SemaphoreType.DMA,
        ])
    def k(idx_hbm, table_hbm, out_hbm, idx_v,
          b0, b1, b2, b3, g0, g1, g2, g3, w0, w1, w2, w3):
        wid = lax.axis_index("s") * _NCORE + lax.axis_index("c")
        p0 = wid * rows_per_w
        s_slot = wid // 2
        c_base = (wid % 2) * 2048
        pltpu.sync_copy(idx_hbm.at[pl.ds(p0, rows_per_w)], idx_v)
        bufs = (b0, b1, b2, b3)
        gsems = (g0, g1, g2, g3)
        wsems = (w0, w1, w2, w3)
        nb = 4
        gcp = [None] * nb
        wcp = [None] * nb
        for j in range(rows_per_w + 1):
            if j < rows_per_w:
                b = j % nb
                if wcp[b] is not None:
                    wcp[b].wait()
                gcp[b] = pltpu.async_copy(table_hbm.at[idx_v.at[j]],
                                          bufs[b], gsems[b])
            if j >= 1:
                b = (j - 1) % nb
                gcp[b].wait()
                wcp[b] = pltpu.async_copy(
                    bufs[b],
                    out_hbm.at[pl.ds(c_base + (j - 1) * 128, 128),
                               pl.ds(s_slot * D_VEC, D_VEC)],
                    wsems[b])
        for b in range(nb):
            if wcp[b] is not None:
                wcp[b].wait()

    return k(idx2d, v)


def _mlp_body(p_ref, n_ref, c_ref, w1_ref, wc_ref, b1_ref, w2_ref, b2_ref,
              o_ref):
    ns = n_ref[...]
    si = lax.broadcasted_iota(jnp.int32, (_MLP_C, K_SET * D_VEC), 1) // D_VEC
    x = p_ref[...] * jnp.where(si < ns, 1.0, 0.0)
    h = jnp.dot(x, w1_ref[...], preferred_element_type=jnp.float32)
    h = h + jnp.log1p(c_ref[...]) * wc_ref[...] + b1_ref[...]
    act = 0.5 * h * (1.0 + lax.erf(h * np.float32(1.0 / np.sqrt(2.0))))
    o_ref[...] = (jnp.dot(act, w2_ref[...],
                          preferred_element_type=jnp.float32) + b2_ref[...])


def _mlp(packed, nsel_c, cnt_c, W1m, w1c, b1r, W2, b2r):
    grid = N_CH // _MLP_C
    return pl.pallas_call(
        _mlp_body,
        grid=(grid,),
        in_specs=[
            pl.BlockSpec((_MLP_C, K_SET * D_VEC), lambda b: (b, 0)),
            pl.BlockSpec((_MLP_C, 1), lambda b: (b, 0)),
            pl.BlockSpec((_MLP_C, 1), lambda b: (b, 0)),
            pl.BlockSpec((K_SET * D_VEC, HID), lambda b: (0, 0)),
            pl.BlockSpec((1, HID), lambda b: (0, 0)),
            pl.BlockSpec((1, HID), lambda b: (0, 0)),
            pl.BlockSpec((HID, D_VEC), lambda b: (0, 0)),
            pl.BlockSpec((1, D_VEC), lambda b: (0, 0)),
        ],
        out_specs=pl.BlockSpec((_MLP_C, D_VEC), lambda b: (b, 0)),
        out_shape=jax.ShapeDtypeStruct((N_CH, D_VEC), jnp.float32),
    )(packed, nsel_c, cnt_c, W1m, w1c, b1r, W2, b2r)


def kernel(v, batch_idx, mask, count, rank_scores, W1, b1, W2, b2):
    sT = rank_scores.T
    mT = mask.T.astype(jnp.float32)
    iT = batch_idx.astype(jnp.float32).T
    gT, nselT = _select(sT, mT, iT)
    idx2d = gT.reshape((K_SET * N_CH) // 128, 128)
    packed = _gather(idx2d, v)
    nsel_c = nselT.reshape(N_CH, 1)
    cnt_c = count.reshape(N_CH, 1)
    w1c = W1[K_SET * D_VEC:].reshape(1, HID)
    return _mlp(packed, nsel_c, cnt_c, W1[:K_SET * D_VEC], w1c,
                b1.reshape(1, HID), W2, b2.reshape(1, D_VEC))
